# baseline (speedup 1.0000x reference)
import jax
import jax.numpy as jnp
from jax.experimental import pallas as pl
from jax.experimental.pallas import tpu as pltpu

B, L, D, H, NL = 1024, 200, 128, 8, 2
HD = D // H
BB = 2
CLAMP = 50.0
BUCKETS = (32, 64, 96, 128, 160, 200)


def _ln(x, g, b, eps=1e-5):
    m = jnp.mean(x, axis=-1, keepdims=True)
    xc = x - m
    v = jnp.mean(xc * xc, axis=-1, keepdims=True)
    return xc * jax.lax.rsqrt(v + eps) * g + b


def _pipeline(Lb, embs, slens, pos_ref, ln0g_ref, ln0b_ref, WqkvT_ref,
              bqkv_ref, WoT_ref, bo_ref, ln1g_ref, ln1b_ref, W1T_ref,
              b1_ref, W2T_ref, b2_ref, ln2g_ref, ln2b_ref):
    x = jnp.concatenate(
        [embs[s][0, :Lb, :] + pos_ref[:Lb, :] for s in range(BB)], axis=0)
    x = _ln(x, ln0g_ref[0], ln0b_ref[0])

    rows = jax.lax.broadcasted_iota(jnp.int32, (Lb, Lb), 0)
    cols = jax.lax.broadcasted_iota(jnp.int32, (Lb, Lb), 1)

    lane = jax.lax.broadcasted_iota(jnp.int32, (1, D), 1) // HD
    lmasks = [jnp.where(lane == h, jnp.float32(1.0), jnp.float32(0.0))
              for h in range(H)]
    E8 = jnp.concatenate(lmasks, axis=0)

    i = 0
    qkv = jnp.dot(x, WqkvT_ref[i], preferred_element_type=jnp.float32)
    qkv = qkv + bqkv_ref[i]
    outs = []
    for s in range(BB):
        maskT = jnp.where((rows <= cols) & (rows < slens[s]),
                          jnp.float32(1.0), jnp.float32(0.0))
        q = qkv[s * Lb:(s + 1) * Lb, :D]
        k = qkv[s * Lb:(s + 1) * Lb, D:2 * D]
        v = qkv[s * Lb:(s + 1) * Lb, 2 * D:]
        KS = jnp.concatenate([k * m for m in lmasks], axis=0)
        VS = jnp.concatenate([v * m for m in lmasks], axis=0)
        sT = jnp.dot(KS, q.T, preferred_element_type=jnp.float32)
        e = jnp.exp(jnp.minimum(sT, CLAMP))
        e = e * jnp.concatenate([maskT] * H, axis=0)
        sums = jnp.sum(e.reshape(H, Lb, Lb), axis=1)
        o = jax.lax.dot_general(e, VS, (((0,), (0,)), ((), ())),
                                preferred_element_type=jnp.float32)
        rexp = jnp.dot((1.0 / sums).T, E8,
                       preferred_element_type=jnp.float32)
        outs.append(o * rexp)
    o = jnp.concatenate(outs, axis=0)
    o = jnp.dot(o, WoT_ref[i], preferred_element_type=jnp.float32)
    o = o + bo_ref[i]
    x = _ln(x + o, ln1g_ref[i], ln1b_ref[i])
    f = jnp.dot(x, W1T_ref[i], preferred_element_type=jnp.float32)
    f = jnp.maximum(f + b1_ref[i], 0.0)
    f = jnp.dot(f, W2T_ref[i], preferred_element_type=jnp.float32)
    f = f + b2_ref[i]
    x = _ln(x + f, ln2g_ref[i], ln2b_ref[i])

    i = 1
    gcol = jax.lax.broadcasted_iota(jnp.int32, (BB, BB * Lb), 1)
    lo = jnp.concatenate(
        [jnp.full((1, 1), s * Lb, jnp.int32) for s in range(BB)], axis=0)
    hi = jnp.concatenate(
        [jnp.full((1, 1), s * Lb + slens[s], jnp.int32) for s in range(BB)],
        axis=0)
    sel = jnp.where(gcol == hi - 1, jnp.float32(1.0), jnp.float32(0.0))
    xl = jnp.dot(sel, x, preferred_element_type=jnp.float32)

    kv = jnp.dot(x, WqkvT_ref[i][:, D:],
                 preferred_element_type=jnp.float32) + bqkv_ref[i][D:]
    ql = jnp.dot(xl, WqkvT_ref[i][:, :D],
                 preferred_element_type=jnp.float32) + bqkv_ref[i][:D]
    mask1 = jnp.where((gcol >= lo) & (gcol < hi), jnp.float32(1.0),
                      jnp.float32(0.0))
    QS = jnp.concatenate([ql * m for m in lmasks], axis=0)
    sh = jax.lax.dot_general(
        QS, kv[:, :D], (((1,), (1,)), ((), ())),
        preferred_element_type=jnp.float32)
    e = jnp.exp(jnp.minimum(sh, CLAMP))
    e = e * jnp.concatenate([mask1] * H, axis=0)
    r = 1.0 / jnp.sum(e, axis=-1, keepdims=True)
    o32 = jnp.dot(e, kv[:, D:], preferred_element_type=jnp.float32)
    E32 = jnp.concatenate(
        [jnp.concatenate([m] * BB, axis=0) for m in lmasks], axis=0)
    o32 = o32 * r * E32
    o = jnp.sum(o32.reshape(H, BB, D), axis=0)
    o = jnp.dot(o, WoT_ref[i], preferred_element_type=jnp.float32)
    o = o + bo_ref[i]
    xl = _ln(xl + o, ln1g_ref[i], ln1b_ref[i])
    f = jnp.dot(xl, W1T_ref[i], preferred_element_type=jnp.float32)
    f = jnp.maximum(f + b1_ref[i], 0.0)
    f = jnp.dot(f, W2T_ref[i], preferred_element_type=jnp.float32)
    f = f + b2_ref[i]
    xl = _ln(xl + f, ln2g_ref[i], ln2b_ref[i])
    return xl


def _fwd(perm_ref, seq_len_ref, *refs):
    embs = refs[:BB]
    (pos_ref, ln0g_ref, ln0b_ref, WqkvT_ref, bqkv_ref, WoT_ref, bo_ref,
     ln1g_ref, ln1b_ref, W1T_ref, b1_ref, W2T_ref, b2_ref, ln2g_ref,
     ln2b_ref, out_ref) = refs[BB:]
    pid = pl.program_id(0)
    slens = [seq_len_ref[perm_ref[pid * BB + s]] for s in range(BB)]
    maxlen = slens[0]
    for s in range(1, BB):
        maxlen = jnp.maximum(maxlen, slens[s])
    wargs = (pos_ref, ln0g_ref, ln0b_ref, WqkvT_ref, bqkv_ref, WoT_ref,
             bo_ref, ln1g_ref, ln1b_ref, W1T_ref, b1_ref, W2T_ref, b2_ref,
             ln2g_ref, ln2b_ref)
    prev = 0
    for Lb in BUCKETS:
        @pl.when((maxlen > prev) & (maxlen <= Lb))
        def _(Lb=Lb):
            out_ref[...] = _pipeline(Lb, embs, slens, *wargs).reshape(
                BB, 1, D)
        prev = Lb


def kernel(seq_emb, seq_len, pos_table, ln0_g, ln0_b, Wqkv, bqkv, Wo, bo,
           ln1_g, ln1_b, W1, b1, W2, b2, ln2_g, ln2_b):
    WqkvT = jnp.transpose(Wqkv, (0, 2, 1))
    scale = jnp.concatenate(
        [jnp.full((1, 1, D), 0.25, jnp.float32),
         jnp.ones((1, 1, 2 * D), jnp.float32)], axis=-1)
    WqkvT = WqkvT * scale
    bqkv = bqkv * scale[0]
    WoT = jnp.transpose(Wo, (0, 2, 1))
    W1T = jnp.transpose(W1, (0, 2, 1))
    W2T = jnp.transpose(W2, (0, 2, 1))
    ln0_g = ln0_g.reshape(1, D)
    ln0_b = ln0_b.reshape(1, D)
    perm = jnp.argsort(seq_len).astype(jnp.int32)

    full = lambda *shape: pl.BlockSpec(shape, lambda b, p, sl: (0,) * len(shape))
    emb_spec = lambda s: pl.BlockSpec(
        (1, L, D), lambda b, p, sl: (p[b * BB + s], 0, 0))
    grid_spec = pltpu.PrefetchScalarGridSpec(
        num_scalar_prefetch=2,
        grid=(B // BB,),
        in_specs=[
            *[emb_spec(s) for s in range(BB)],
            full(L, D),
            full(1, D),
            full(1, D),
            full(NL, D, 3 * D),
            full(NL, 3 * D),
            full(NL, D, D),
            full(NL, D),
            full(NL, D),
            full(NL, D),
            full(NL, D, 4 * D),
            full(NL, 4 * D),
            full(NL, 4 * D, D),
            full(NL, D),
            full(NL, D),
            full(NL, D),
        ],
        out_specs=pl.BlockSpec((BB, 1, D), lambda b, p, sl: (b, 0, 0)),
    )
    out = pl.pallas_call(
        _fwd,
        grid_spec=grid_spec,
        out_shape=jax.ShapeDtypeStruct((B, 1, D), jnp.float32),
        compiler_params=pltpu.CompilerParams(
            dimension_semantics=("arbitrary",)),
    )(perm, seq_len, *([seq_emb] * BB), pos_table,
      ln0_g, ln0_b, WqkvT, bqkv, WoT, bo, ln1_g, ln1_b, W1T, b1, W2T, b2,
      ln2_g, ln2_b)
    return out.reshape(B, D)[jnp.argsort(perm)]


# speedup vs baseline: 1.5068x; 1.5068x over previous
import jax
import jax.numpy as jnp
from jax.experimental import pallas as pl
from jax.experimental.pallas import tpu as pltpu

B, L, D, H, NL = 1024, 200, 128, 8, 2
HD = D // H
BB = 4
CLAMP = 50.0
BUCKETS = (32, 64, 96, 128, 160, 200)


def _ln(x, g, b, eps=1e-5):
    m = jnp.mean(x, axis=-1, keepdims=True)
    xc = x - m
    v = jnp.mean(xc * xc, axis=-1, keepdims=True)
    return xc * jax.lax.rsqrt(v + eps) * g + b


def _pipeline(Lb, embs, slens, pos_ref, ln0g_ref, ln0b_ref, WqkvT_ref,
              bqkv_ref, WoT_ref, bo_ref, ln1g_ref, ln1b_ref, W1T_ref,
              b1_ref, W2T_ref, b2_ref, ln2g_ref, ln2b_ref):
    x = jnp.concatenate(
        [embs[s][0, :Lb, :] + pos_ref[:Lb, :] for s in range(BB)], axis=0)
    x = _ln(x, ln0g_ref[0], ln0b_ref[0])

    rows = jax.lax.broadcasted_iota(jnp.int32, (Lb, Lb), 0)
    cols = jax.lax.broadcasted_iota(jnp.int32, (Lb, Lb), 1)

    lane = jax.lax.broadcasted_iota(jnp.int32, (1, D), 1) // HD
    lmasks = [jnp.where(lane == h, jnp.float32(1.0), jnp.float32(0.0))
              for h in range(H)]
    E8 = jnp.concatenate(lmasks, axis=0)

    i = 0
    qkv = jnp.dot(x, WqkvT_ref[i], preferred_element_type=jnp.float32)
    qkv = qkv + bqkv_ref[i]
    outs = []
    for s in range(BB):
        maskT = jnp.where((rows <= cols) & (rows < slens[s]),
                          jnp.float32(1.0), jnp.float32(0.0))
        q = qkv[s * Lb:(s + 1) * Lb, :D]
        k = qkv[s * Lb:(s + 1) * Lb, D:2 * D]
        v = qkv[s * Lb:(s + 1) * Lb, 2 * D:]
        KS = jnp.concatenate([k * m for m in lmasks], axis=0)
        VS = jnp.concatenate([v * m for m in lmasks], axis=0)
        sT = jnp.dot(KS, q.T, preferred_element_type=jnp.float32)
        e = jnp.exp(jnp.minimum(sT, CLAMP))
        e = e * jnp.concatenate([maskT] * H, axis=0)
        sums = jnp.sum(e.reshape(H, Lb, Lb), axis=1)
        o = jax.lax.dot_general(e, VS, (((0,), (0,)), ((), ())),
                                preferred_element_type=jnp.float32)
        rexp = jnp.dot((1.0 / sums).T, E8,
                       preferred_element_type=jnp.float32)
        outs.append(o * rexp)
    o = jnp.concatenate(outs, axis=0)
    o = jnp.dot(o, WoT_ref[i], preferred_element_type=jnp.float32)
    o = o + bo_ref[i]
    x = _ln(x + o, ln1g_ref[i], ln1b_ref[i])
    f = jnp.dot(x, W1T_ref[i], preferred_element_type=jnp.float32)
    f = jnp.maximum(f + b1_ref[i], 0.0)
    f = jnp.dot(f, W2T_ref[i], preferred_element_type=jnp.float32)
    f = f + b2_ref[i]
    x = _ln(x + f, ln2g_ref[i], ln2b_ref[i])

    i = 1
    gcol = jax.lax.broadcasted_iota(jnp.int32, (BB, BB * Lb), 1)
    lo = jnp.concatenate(
        [jnp.full((1, 1), s * Lb, jnp.int32) for s in range(BB)], axis=0)
    hi = jnp.concatenate(
        [jnp.full((1, 1), s * Lb + slens[s], jnp.int32) for s in range(BB)],
        axis=0)
    sel = jnp.where(gcol == hi - 1, jnp.float32(1.0), jnp.float32(0.0))
    xl = jnp.dot(sel, x, preferred_element_type=jnp.float32)

    kv = jnp.dot(x, WqkvT_ref[i][:, D:],
                 preferred_element_type=jnp.float32) + bqkv_ref[i][D:]
    ql = jnp.dot(xl, WqkvT_ref[i][:, :D],
                 preferred_element_type=jnp.float32) + bqkv_ref[i][:D]
    mask1 = jnp.where((gcol >= lo) & (gcol < hi), jnp.float32(1.0),
                      jnp.float32(0.0))
    QS = jnp.concatenate([ql * m for m in lmasks], axis=0)
    sh = jax.lax.dot_general(
        QS, kv[:, :D], (((1,), (1,)), ((), ())),
        preferred_element_type=jnp.float32)
    e = jnp.exp(jnp.minimum(sh, CLAMP))
    e = e * jnp.concatenate([mask1] * H, axis=0)
    r = 1.0 / jnp.sum(e, axis=-1, keepdims=True)
    o32 = jnp.dot(e, kv[:, D:], preferred_element_type=jnp.float32)
    E32 = jnp.concatenate(
        [jnp.concatenate([m] * BB, axis=0) for m in lmasks], axis=0)
    o32 = o32 * r * E32
    o = jnp.sum(o32.reshape(H, BB, D), axis=0)
    o = jnp.dot(o, WoT_ref[i], preferred_element_type=jnp.float32)
    o = o + bo_ref[i]
    xl = _ln(xl + o, ln1g_ref[i], ln1b_ref[i])
    f = jnp.dot(xl, W1T_ref[i], preferred_element_type=jnp.float32)
    f = jnp.maximum(f + b1_ref[i], 0.0)
    f = jnp.dot(f, W2T_ref[i], preferred_element_type=jnp.float32)
    f = f + b2_ref[i]
    xl = _ln(xl + f, ln2g_ref[i], ln2b_ref[i])
    return xl


def _fwd(perm_ref, seq_len_ref, *refs):
    embs = refs[:BB]
    (pos_ref, ln0g_ref, ln0b_ref, WqkvT_ref, bqkv_ref, WoT_ref, bo_ref,
     ln1g_ref, ln1b_ref, W1T_ref, b1_ref, W2T_ref, b2_ref, ln2g_ref,
     ln2b_ref, out_ref) = refs[BB:]
    pid = pl.program_id(0)
    slens = [seq_len_ref[perm_ref[pid * BB + s]] for s in range(BB)]
    maxlen = slens[0]
    for s in range(1, BB):
        maxlen = jnp.maximum(maxlen, slens[s])
    wargs = (pos_ref, ln0g_ref, ln0b_ref, WqkvT_ref, bqkv_ref, WoT_ref,
             bo_ref, ln1g_ref, ln1b_ref, W1T_ref, b1_ref, W2T_ref, b2_ref,
             ln2g_ref, ln2b_ref)
    prev = 0
    for Lb in BUCKETS:
        @pl.when((maxlen > prev) & (maxlen <= Lb))
        def _(Lb=Lb):
            out_ref[...] = _pipeline(Lb, embs, slens, *wargs).reshape(
                BB, 1, D)
        prev = Lb


def kernel(seq_emb, seq_len, pos_table, ln0_g, ln0_b, Wqkv, bqkv, Wo, bo,
           ln1_g, ln1_b, W1, b1, W2, b2, ln2_g, ln2_b):
    WqkvT = jnp.transpose(Wqkv, (0, 2, 1))
    scale = jnp.concatenate(
        [jnp.full((1, 1, D), 0.25, jnp.float32),
         jnp.ones((1, 1, 2 * D), jnp.float32)], axis=-1)
    WqkvT = WqkvT * scale
    bqkv = bqkv * scale[0]
    WoT = jnp.transpose(Wo, (0, 2, 1))
    W1T = jnp.transpose(W1, (0, 2, 1))
    W2T = jnp.transpose(W2, (0, 2, 1))
    ln0_g = ln0_g.reshape(1, D)
    ln0_b = ln0_b.reshape(1, D)
    perm = jnp.argsort(seq_len).astype(jnp.int32)

    full = lambda *shape: pl.BlockSpec(shape, lambda b, p, sl: (0,) * len(shape))
    emb_spec = lambda s: pl.BlockSpec(
        (1, L, D), lambda b, p, sl: (p[b * BB + s], 0, 0))
    grid_spec = pltpu.PrefetchScalarGridSpec(
        num_scalar_prefetch=2,
        grid=(B // BB,),
        in_specs=[
            *[emb_spec(s) for s in range(BB)],
            full(L, D),
            full(1, D),
            full(1, D),
            full(NL, D, 3 * D),
            full(NL, 3 * D),
            full(NL, D, D),
            full(NL, D),
            full(NL, D),
            full(NL, D),
            full(NL, D, 4 * D),
            full(NL, 4 * D),
            full(NL, 4 * D, D),
            full(NL, D),
            full(NL, D),
            full(NL, D),
        ],
        out_specs=pl.BlockSpec((BB, 1, D), lambda b, p, sl: (b, 0, 0)),
    )
    out = pl.pallas_call(
        _fwd,
        grid_spec=grid_spec,
        out_shape=jax.ShapeDtypeStruct((B, 1, D), jnp.float32),
        compiler_params=pltpu.CompilerParams(
            dimension_semantics=("parallel",)),
    )(perm, seq_len, *([seq_emb] * BB), pos_table,
      ln0_g, ln0_b, WqkvT, bqkv, WoT, bo, ln1_g, ln1_b, W1T, b1, W2T, b2,
      ln2_g, ln2_b)
    return out.reshape(B, D)[jnp.argsort(perm)]


# speedup vs baseline: 1.5250x; 1.0121x over previous
import jax
import jax.numpy as jnp
from jax.experimental import pallas as pl
from jax.experimental.pallas import tpu as pltpu

B, L, D, H, NL = 1024, 200, 128, 8, 2
HD = D // H
BB = 4
CLAMP = 50.0
BUCKETS = (24, 48, 72, 96, 128, 160, 200)


def _ln(x, g, b, eps=1e-5):
    m = jnp.mean(x, axis=-1, keepdims=True)
    xc = x - m
    v = jnp.mean(xc * xc, axis=-1, keepdims=True)
    return xc * jax.lax.rsqrt(v + eps) * g + b


def _pipeline(Lb, embs, slens, pos_ref, ln0g_ref, ln0b_ref, WqkvT_ref,
              bqkv_ref, WoT_ref, bo_ref, ln1g_ref, ln1b_ref, W1T_ref,
              b1_ref, W2T_ref, b2_ref, ln2g_ref, ln2b_ref):
    x = jnp.concatenate(
        [embs[s][0, :Lb, :] + pos_ref[:Lb, :] for s in range(BB)], axis=0)
    x = _ln(x, ln0g_ref[0], ln0b_ref[0])

    rows = jax.lax.broadcasted_iota(jnp.int32, (Lb, Lb), 0)
    cols = jax.lax.broadcasted_iota(jnp.int32, (Lb, Lb), 1)

    lane = jax.lax.broadcasted_iota(jnp.int32, (1, D), 1) // HD
    lmasks = [jnp.where(lane == h, jnp.float32(1.0), jnp.float32(0.0))
              for h in range(H)]
    E8 = jnp.concatenate(lmasks, axis=0)

    i = 0
    qkv = jnp.dot(x, WqkvT_ref[i], preferred_element_type=jnp.float32)
    qkv = qkv + bqkv_ref[i]
    outs = []
    for s in range(BB):
        maskT = jnp.where((rows <= cols) & (rows < slens[s]),
                          jnp.float32(1.0), jnp.float32(0.0))
        q = qkv[s * Lb:(s + 1) * Lb, :D]
        k = qkv[s * Lb:(s + 1) * Lb, D:2 * D]
        v = qkv[s * Lb:(s + 1) * Lb, 2 * D:]
        KS = jnp.concatenate([k * m for m in lmasks], axis=0)
        VS = jnp.concatenate([v * m for m in lmasks], axis=0)
        sT = jnp.dot(KS, q.T, preferred_element_type=jnp.float32)
        e = jnp.exp(jnp.minimum(sT, CLAMP))
        e = e * jnp.concatenate([maskT] * H, axis=0)
        sums = jnp.sum(e.reshape(H, Lb, Lb), axis=1)
        o = jax.lax.dot_general(e, VS, (((0,), (0,)), ((), ())),
                                preferred_element_type=jnp.float32)
        rexp = jnp.dot((1.0 / sums).T, E8,
                       preferred_element_type=jnp.float32)
        outs.append(o * rexp)
    o = jnp.concatenate(outs, axis=0)
    o = jnp.dot(o, WoT_ref[i], preferred_element_type=jnp.float32)
    o = o + bo_ref[i]
    x = _ln(x + o, ln1g_ref[i], ln1b_ref[i])
    f = jnp.dot(x, W1T_ref[i], preferred_element_type=jnp.float32)
    f = jnp.maximum(f + b1_ref[i], 0.0)
    f = jnp.dot(f, W2T_ref[i], preferred_element_type=jnp.float32)
    f = f + b2_ref[i]
    x = _ln(x + f, ln2g_ref[i], ln2b_ref[i])

    i = 1
    gcol = jax.lax.broadcasted_iota(jnp.int32, (BB, BB * Lb), 1)
    lo = jnp.concatenate(
        [jnp.full((1, 1), s * Lb, jnp.int32) for s in range(BB)], axis=0)
    hi = jnp.concatenate(
        [jnp.full((1, 1), s * Lb + slens[s], jnp.int32) for s in range(BB)],
        axis=0)
    sel = jnp.where(gcol == hi - 1, jnp.float32(1.0), jnp.float32(0.0))
    xl = jnp.dot(sel, x, preferred_element_type=jnp.float32)

    kv = jnp.dot(x, WqkvT_ref[i][:, D:],
                 preferred_element_type=jnp.float32) + bqkv_ref[i][D:]
    ql = jnp.dot(xl, WqkvT_ref[i][:, :D],
                 preferred_element_type=jnp.float32) + bqkv_ref[i][:D]
    mask1 = jnp.where((gcol >= lo) & (gcol < hi), jnp.float32(1.0),
                      jnp.float32(0.0))
    QS = jnp.concatenate([ql * m for m in lmasks], axis=0)
    sh = jax.lax.dot_general(
        QS, kv[:, :D], (((1,), (1,)), ((), ())),
        preferred_element_type=jnp.float32)
    e = jnp.exp(jnp.minimum(sh, CLAMP))
    e = e * jnp.concatenate([mask1] * H, axis=0)
    r = 1.0 / jnp.sum(e, axis=-1, keepdims=True)
    o32 = jnp.dot(e, kv[:, D:], preferred_element_type=jnp.float32)
    E32 = jnp.concatenate(
        [jnp.concatenate([m] * BB, axis=0) for m in lmasks], axis=0)
    o32 = o32 * r * E32
    o = jnp.sum(o32.reshape(H, BB, D), axis=0)
    o = jnp.dot(o, WoT_ref[i], preferred_element_type=jnp.float32)
    o = o + bo_ref[i]
    xl = _ln(xl + o, ln1g_ref[i], ln1b_ref[i])
    f = jnp.dot(xl, W1T_ref[i], preferred_element_type=jnp.float32)
    f = jnp.maximum(f + b1_ref[i], 0.0)
    f = jnp.dot(f, W2T_ref[i], preferred_element_type=jnp.float32)
    f = f + b2_ref[i]
    xl = _ln(xl + f, ln2g_ref[i], ln2b_ref[i])
    return xl


def _fwd(perm_ref, seq_len_ref, *refs):
    embs = refs[:BB]
    (pos_ref, ln0g_ref, ln0b_ref, WqkvT_ref, bqkv_ref, WoT_ref, bo_ref,
     ln1g_ref, ln1b_ref, W1T_ref, b1_ref, W2T_ref, b2_ref, ln2g_ref,
     ln2b_ref, out_ref) = refs[BB:]
    pid = pl.program_id(0)
    slens = [seq_len_ref[perm_ref[pid * BB + s]] for s in range(BB)]
    maxlen = slens[0]
    for s in range(1, BB):
        maxlen = jnp.maximum(maxlen, slens[s])
    wargs = (pos_ref, ln0g_ref, ln0b_ref, WqkvT_ref, bqkv_ref, WoT_ref,
             bo_ref, ln1g_ref, ln1b_ref, W1T_ref, b1_ref, W2T_ref, b2_ref,
             ln2g_ref, ln2b_ref)
    prev = 0
    for Lb in BUCKETS:
        @pl.when((maxlen > prev) & (maxlen <= Lb))
        def _(Lb=Lb):
            out_ref[...] = _pipeline(Lb, embs, slens, *wargs).reshape(
                BB, 1, D)
        prev = Lb


def kernel(seq_emb, seq_len, pos_table, ln0_g, ln0_b, Wqkv, bqkv, Wo, bo,
           ln1_g, ln1_b, W1, b1, W2, b2, ln2_g, ln2_b):
    WqkvT = jnp.transpose(Wqkv, (0, 2, 1))
    scale = jnp.concatenate(
        [jnp.full((1, 1, D), 0.25, jnp.float32),
         jnp.ones((1, 1, 2 * D), jnp.float32)], axis=-1)
    WqkvT = WqkvT * scale
    bqkv = bqkv * scale[0]
    WoT = jnp.transpose(Wo, (0, 2, 1))
    W1T = jnp.transpose(W1, (0, 2, 1))
    W2T = jnp.transpose(W2, (0, 2, 1))
    ln0_g = ln0_g.reshape(1, D)
    ln0_b = ln0_b.reshape(1, D)
    perm = jnp.argsort(seq_len).astype(jnp.int32)

    full = lambda *shape: pl.BlockSpec(shape, lambda b, p, sl: (0,) * len(shape))
    emb_spec = lambda s: pl.BlockSpec(
        (1, L, D), lambda b, p, sl: (p[b * BB + s], 0, 0))
    grid_spec = pltpu.PrefetchScalarGridSpec(
        num_scalar_prefetch=2,
        grid=(B // BB,),
        in_specs=[
            *[emb_spec(s) for s in range(BB)],
            full(L, D),
            full(1, D),
            full(1, D),
            full(NL, D, 3 * D),
            full(NL, 3 * D),
            full(NL, D, D),
            full(NL, D),
            full(NL, D),
            full(NL, D),
            full(NL, D, 4 * D),
            full(NL, 4 * D),
            full(NL, 4 * D, D),
            full(NL, D),
            full(NL, D),
            full(NL, D),
        ],
        out_specs=pl.BlockSpec((BB, 1, D), lambda b, p, sl: (b, 0, 0)),
    )
    out = pl.pallas_call(
        _fwd,
        grid_spec=grid_spec,
        out_shape=jax.ShapeDtypeStruct((B, 1, D), jnp.float32),
        compiler_params=pltpu.CompilerParams(
            dimension_semantics=("parallel",)),
    )(perm, seq_len, *([seq_emb] * BB), pos_table,
      ln0_g, ln0_b, WqkvT, bqkv, WoT, bo, ln1_g, ln1_b, W1T, b1, W2T, b2,
      ln2_g, ln2_b)
    return out.reshape(B, D)[jnp.argsort(perm)]


# speedup vs baseline: 1.5549x; 1.0196x over previous
import jax
import jax.numpy as jnp
from jax.experimental import pallas as pl
from jax.experimental.pallas import tpu as pltpu

B, L, D, H, NL = 1024, 200, 128, 8, 2
HD = D // H
BB = 4
CLAMP = 50.0
BUCKETS = (24, 48, 72, 96, 128, 160, 200)


def _ln(x, g, b, eps=1e-5):
    m = jnp.mean(x, axis=-1, keepdims=True)
    xc = x - m
    v = jnp.mean(xc * xc, axis=-1, keepdims=True)
    return xc * jax.lax.rsqrt(v + eps) * g + b


def _pipeline(Lb, embs, slens, pos_ref, ln0g_ref, ln0b_ref, WqkvT_ref,
              bqkv_ref, WoT_ref, bo_ref, ln1g_ref, ln1b_ref, W1T_ref,
              b1_ref, W2T_ref, b2_ref, ln2g_ref, ln2b_ref):
    x = jnp.concatenate(
        [embs[s][0, :Lb, :] + pos_ref[:Lb, :] for s in range(BB)], axis=0)
    x = _ln(x, ln0g_ref[0], ln0b_ref[0])

    rows = jax.lax.broadcasted_iota(jnp.int32, (Lb, Lb), 0)
    cols = jax.lax.broadcasted_iota(jnp.int32, (Lb, Lb), 1)

    lane = jax.lax.broadcasted_iota(jnp.int32, (1, D), 1) // HD
    lmasks = [jnp.where(lane == h, jnp.float32(1.0), jnp.float32(0.0))
              for h in range(H)]
    E8 = jnp.concatenate(lmasks, axis=0)

    i = 0
    qkv = jnp.dot(x, WqkvT_ref[i], preferred_element_type=jnp.float32)
    qkv = qkv + bqkv_ref[i]
    outs = []
    for s in range(BB):
        maskT = jnp.where((rows <= cols) & (rows < slens[s]),
                          jnp.float32(1.0), jnp.float32(0.0))
        q = qkv[s * Lb:(s + 1) * Lb, :D]
        k = qkv[s * Lb:(s + 1) * Lb, D:2 * D]
        v = qkv[s * Lb:(s + 1) * Lb, 2 * D:]
        KS = jnp.concatenate([k * m for m in lmasks], axis=0)
        VS = jnp.concatenate([v * m for m in lmasks], axis=0)
        sT = jnp.dot(KS, q.T, preferred_element_type=jnp.float32)
        e = jnp.exp(jnp.minimum(sT, CLAMP))
        e = e * jnp.concatenate([maskT] * H, axis=0)
        sums = jnp.sum(e.reshape(H, Lb, Lb), axis=1)
        o = jax.lax.dot_general(e.astype(jnp.bfloat16),
                                VS.astype(jnp.bfloat16),
                                (((0,), (0,)), ((), ())),
                                preferred_element_type=jnp.float32)
        rexp = jnp.dot((1.0 / sums).T, E8,
                       preferred_element_type=jnp.float32)
        outs.append(o * rexp)
    o = jnp.concatenate(outs, axis=0)
    o = jnp.dot(o, WoT_ref[i], preferred_element_type=jnp.float32)
    o = o + bo_ref[i]
    x = _ln(x + o, ln1g_ref[i], ln1b_ref[i])
    f = jnp.dot(x, W1T_ref[i], preferred_element_type=jnp.float32)
    f = jnp.maximum(f + b1_ref[i], 0.0)
    f = jnp.dot(f, W2T_ref[i], preferred_element_type=jnp.float32)
    f = f + b2_ref[i]
    x = _ln(x + f, ln2g_ref[i], ln2b_ref[i])

    i = 1
    gcol = jax.lax.broadcasted_iota(jnp.int32, (BB, BB * Lb), 1)
    lo = jnp.concatenate(
        [jnp.full((1, 1), s * Lb, jnp.int32) for s in range(BB)], axis=0)
    hi = jnp.concatenate(
        [jnp.full((1, 1), s * Lb + slens[s], jnp.int32) for s in range(BB)],
        axis=0)
    sel = jnp.where(gcol == hi - 1, jnp.float32(1.0), jnp.float32(0.0))
    xl = jnp.dot(sel, x, preferred_element_type=jnp.float32)

    kv = jnp.dot(x, WqkvT_ref[i][:, D:],
                 preferred_element_type=jnp.float32) + bqkv_ref[i][D:]
    ql = jnp.dot(xl, WqkvT_ref[i][:, :D],
                 preferred_element_type=jnp.float32) + bqkv_ref[i][:D]
    mask1 = jnp.where((gcol >= lo) & (gcol < hi), jnp.float32(1.0),
                      jnp.float32(0.0))
    QS = jnp.concatenate([ql * m for m in lmasks], axis=0)
    sh = jax.lax.dot_general(
        QS, kv[:, :D], (((1,), (1,)), ((), ())),
        preferred_element_type=jnp.float32)
    e = jnp.exp(jnp.minimum(sh, CLAMP))
    e = e * jnp.concatenate([mask1] * H, axis=0)
    r = 1.0 / jnp.sum(e, axis=-1, keepdims=True)
    o32 = jnp.dot(e, kv[:, D:], preferred_element_type=jnp.float32)
    E32 = jnp.concatenate(
        [jnp.concatenate([m] * BB, axis=0) for m in lmasks], axis=0)
    o32 = o32 * r * E32
    o = jnp.sum(o32.reshape(H, BB, D), axis=0)
    o = jnp.dot(o, WoT_ref[i], preferred_element_type=jnp.float32)
    o = o + bo_ref[i]
    xl = _ln(xl + o, ln1g_ref[i], ln1b_ref[i])
    f = jnp.dot(xl, W1T_ref[i], preferred_element_type=jnp.float32)
    f = jnp.maximum(f + b1_ref[i], 0.0)
    f = jnp.dot(f, W2T_ref[i], preferred_element_type=jnp.float32)
    f = f + b2_ref[i]
    xl = _ln(xl + f, ln2g_ref[i], ln2b_ref[i])
    return xl


def _fwd(perm_ref, seq_len_ref, *refs):
    embs = refs[:BB]
    (pos_ref, ln0g_ref, ln0b_ref, WqkvT_ref, bqkv_ref, WoT_ref, bo_ref,
     ln1g_ref, ln1b_ref, W1T_ref, b1_ref, W2T_ref, b2_ref, ln2g_ref,
     ln2b_ref, out_ref) = refs[BB:]
    pid = pl.program_id(0)
    slens = [seq_len_ref[perm_ref[pid * BB + s]] for s in range(BB)]
    maxlen = slens[0]
    for s in range(1, BB):
        maxlen = jnp.maximum(maxlen, slens[s])
    wargs = (pos_ref, ln0g_ref, ln0b_ref, WqkvT_ref, bqkv_ref, WoT_ref,
             bo_ref, ln1g_ref, ln1b_ref, W1T_ref, b1_ref, W2T_ref, b2_ref,
             ln2g_ref, ln2b_ref)
    prev = 0
    for Lb in BUCKETS:
        @pl.when((maxlen > prev) & (maxlen <= Lb))
        def _(Lb=Lb):
            out_ref[...] = _pipeline(Lb, embs, slens, *wargs).reshape(
                BB, 1, D)
        prev = Lb


def kernel(seq_emb, seq_len, pos_table, ln0_g, ln0_b, Wqkv, bqkv, Wo, bo,
           ln1_g, ln1_b, W1, b1, W2, b2, ln2_g, ln2_b):
    WqkvT = jnp.transpose(Wqkv, (0, 2, 1))
    scale = jnp.concatenate(
        [jnp.full((1, 1, D), 0.25, jnp.float32),
         jnp.ones((1, 1, 2 * D), jnp.float32)], axis=-1)
    WqkvT = WqkvT * scale
    bqkv = bqkv * scale[0]
    WoT = jnp.transpose(Wo, (0, 2, 1))
    W1T = jnp.transpose(W1, (0, 2, 1))
    W2T = jnp.transpose(W2, (0, 2, 1))
    ln0_g = ln0_g.reshape(1, D)
    ln0_b = ln0_b.reshape(1, D)
    perm = jnp.argsort(seq_len).astype(jnp.int32)

    full = lambda *shape: pl.BlockSpec(shape, lambda b, p, sl: (0,) * len(shape))
    emb_spec = lambda s: pl.BlockSpec(
        (1, L, D), lambda b, p, sl: (p[b * BB + s], 0, 0))
    grid_spec = pltpu.PrefetchScalarGridSpec(
        num_scalar_prefetch=2,
        grid=(B // BB,),
        in_specs=[
            *[emb_spec(s) for s in range(BB)],
            full(L, D),
            full(1, D),
            full(1, D),
            full(NL, D, 3 * D),
            full(NL, 3 * D),
            full(NL, D, D),
            full(NL, D),
            full(NL, D),
            full(NL, D),
            full(NL, D, 4 * D),
            full(NL, 4 * D),
            full(NL, 4 * D, D),
            full(NL, D),
            full(NL, D),
            full(NL, D),
        ],
        out_specs=pl.BlockSpec((BB, 1, D), lambda b, p, sl: (b, 0, 0)),
    )
    out = pl.pallas_call(
        _fwd,
        grid_spec=grid_spec,
        out_shape=jax.ShapeDtypeStruct((B, 1, D), jnp.float32),
        compiler_params=pltpu.CompilerParams(
            dimension_semantics=("parallel",)),
    )(perm, seq_len, *([seq_emb] * BB), pos_table,
      ln0_g, ln0_b, WqkvT, bqkv, WoT, bo, ln1_g, ln1_b, W1T, b1, W2T, b2,
      ln2_g, ln2_b)
    return out.reshape(B, D)[jnp.argsort(perm)]
